# Initial kernel scaffold; baseline (speedup 1.0000x reference)
import jax
import jax.numpy as jnp
from jax.experimental import pallas as pl


def kernel(embeddings, edge_index, edge_type, comp1, bases1, root1, bias1, comp2, bases2, root2, bias2):
    raise NotImplementedError("write your pallas kernel here")


# speedup vs baseline: 11.3022x; 11.3022x over previous
import jax
import jax.numpy as jnp
from jax import lax
from jax.experimental import pallas as pl
from jax.experimental.pallas import tpu as pltpu
from jax.experimental.pallas import tpu_sc as plsc

N = 10000
E = 320000
R = 8
NBASES = 30
NR = R * N
NT = 16
ET = E // NT
CH = 2000
NCH = ET // CH
CHR = CH // 16
PASSES = ((0, 26624), (26624, 26624), (53248, 26752))
ACC_ROWS = 26752 + 64
ALPHA_LRELU = 0.01


def _sc_compiler_params():
    return pltpu.CompilerParams(use_tc_tiling_on_sc=False)


def _fill_zero_rows(zrow):
    def fz(i, _):
        zrow[i, :] = jnp.zeros((16,), jnp.float32)
        return 0
    lax.fori_loop(0, 125, fz, 0)


def _zero_rows(acc, zrow, base, nrows):
    done = 0
    while done < nrows:
        k = min(125, nrows - done)
        pltpu.sync_copy(zrow.at[pl.ds(0, k)], acc.at[pl.ds(base + done, k)])
        done += k


def _load_seg_ids(ei_hbm, et_hbm, tmpa, tmpb, sidx, t0):
    for j in range(NCH):
        pltpu.sync_copy(ei_hbm.at[pl.ds(E + t0 + j * CH, CH)], tmpa)
        pltpu.sync_copy(et_hbm.at[pl.ds(t0 + j * CH, CH)], tmpb)
        sj = sidx[j]

        def sbody(i, _):
            d = tmpa[pl.ds(i * 16, 16)]
            t = tmpb[pl.ds(i * 16, 16)]
            sj[pl.ds(i * 16, 16)] = d * R + t
            return 0
        lax.fori_loop(0, CHR, sbody, 0)


def _repack_dump(acc, vbuf, obuf, out2d, acc_base, out_base, nrows):
    done = 0
    while done < nrows:
        chunk = min(1000, nrows - done)
        pltpu.sync_copy(acc.at[pl.ds(acc_base + done, chunk)],
                        vbuf.at[pl.ds(0, chunk)])
        orows = chunk // 8

        def rp(q, _):
            for t in range(8):
                obuf[q, pl.ds(t * 16, 16)] = vbuf[q * 8 + t, :]
            return 0
        lax.fori_loop(0, orows, rp, 0)
        pltpu.sync_copy(obuf.at[pl.ds(0, orows)],
                        out2d.at[pl.ds(out_base + done // 8, orows)])
        done += chunk


def _sc_counts():
    mesh = plsc.VectorSubcoreMesh(core_axis_name="c", subcore_axis_name="s")
    scratch = (
        [pltpu.VMEM((CH,), jnp.int32) for _ in range(NCH)]
        + [
            pltpu.VMEM((CH,), jnp.int32),
            pltpu.VMEM((CH,), jnp.int32),
            pltpu.VMEM((CH,), jnp.float32),
            pltpu.VMEM((5008,), jnp.float32),
            pltpu.VMEM((1008,), jnp.float32),
            pltpu.VMEM((125, 128), jnp.float32),
            pltpu.VMEM_SHARED((NR,), jnp.float32),
        ]
    )

    def body(ei_hbm, et_hbm, c_out, *rest):
        sidx = rest[0:NCH]
        tmpa, tmpb, ones, zflat, cbuf, obuf, cacc = rest[NCH:]
        cid = lax.axis_index("c")
        sid = lax.axis_index("s")
        t0 = sid * ET

        @pl.when(cid == 0)
        def _():
            def fo(i, _):
                ones[pl.ds(i * 16, 16)] = jnp.ones((16,), jnp.float32)
                zflat[pl.ds(i * 16, 16)] = jnp.zeros((16,), jnp.float32)
                return 0
            lax.fori_loop(0, CHR, fo, 0)

            def fz2(i, _):
                zflat[pl.ds(i * 16, 16)] = jnp.zeros((16,), jnp.float32)
                return 0
            lax.fori_loop(CHR, 313, fz2, 0)

            for j in range(NCH):
                pltpu.sync_copy(ei_hbm.at[pl.ds(E + t0 + j * CH, CH)], tmpa)
                pltpu.sync_copy(et_hbm.at[pl.ds(t0 + j * CH, CH)], tmpb)
                sj = sidx[j]

                def sbody(i, _):
                    d = tmpa[pl.ds(i * 16, 16)]
                    t = tmpb[pl.ds(i * 16, 16)]
                    sj[pl.ds(i * 16, 16)] = d * R + t
                    return 0
                lax.fori_loop(0, CHR, sbody, 0)

            pltpu.sync_copy(zflat.at[pl.ds(0, 5000)],
                            cacc.at[pl.ds(sid * 5000, 5000)])
            plsc.subcore_barrier()
            for j in range(NCH):
                pltpu.sync_copy(ones, cacc.at[sidx[j]], add=True)
            plsc.subcore_barrier()
            for k in range(5):
                pltpu.sync_copy(cacc.at[pl.ds(sid * 5000 + k * 1000, 1000)],
                                cbuf.at[pl.ds(0, 1000)])

                def ex(q, _):
                    v = cbuf[pl.ds(q * 8, 16)]
                    for r in range(8):
                        obuf[q, pl.ds(r * 16, 16)] = jnp.full(
                            (16,), v[r], jnp.float32)
                    return 0
                lax.fori_loop(0, 125, ex, 0)
                pltpu.sync_copy(obuf, c_out.at[pl.ds(sid * 625 + k * 125,
                                                     125)])

    return pl.kernel(
        body,
        out_type=[jax.ShapeDtypeStruct((N, 128), jnp.float32)],
        mesh=mesh,
        scratch_types=scratch,
        compiler_params=_sc_compiler_params(),
    )


def _sc_segsum(nc):
    mesh = plsc.VectorSubcoreMesh(core_axis_name="c", subcore_axis_name="s")
    scratch = (
        [pltpu.VMEM((CH,), jnp.int32) for _ in range(2 * NCH)]
        + [
            pltpu.VMEM((CH,), jnp.int32),
            pltpu.VMEM((CH,), jnp.int32),
            pltpu.VMEM((CH,), jnp.int32),
            pltpu.VMEM((CH, 16), jnp.float32),
            pltpu.VMEM((125, 16), jnp.float32),
            pltpu.VMEM((125, 128), jnp.float32),
            pltpu.HBM((2, N * 8, 16), jnp.float32),
            pltpu.VMEM_SHARED((ACC_ROWS, 16), jnp.float32),
        ]
    )
    nper = nc // 2

    def body(x128, ei_hbm, et_hbm, s_out, *rest):
        segb = rest[0:NCH]
        gidx = rest[NCH:2 * NCH]
        (scur, tmpa, tmpb, stg, zrow, obuf, xtab, acc) = rest[2 * NCH:]
        cid = lax.axis_index("c")
        sid = lax.axis_index("s")
        t0 = sid * ET
        x_hbm = xtab.at[cid]

        _fill_zero_rows(zrow)

        for k in range(5):
            pltpu.sync_copy(x128.at[pl.ds(sid * 625 + k * 125, 125)], obuf)

            def up(q, _):
                for t in range(8):
                    stg[q * 8 + t, :] = obuf[q, pl.ds(t * 16, 16)]
                return 0
            lax.fori_loop(0, 125, up, 0)
            pltpu.sync_copy(stg.at[pl.ds(0, 1000)],
                            x_hbm.at[pl.ds(sid * 5000 + k * 1000, 1000)])

        _load_seg_ids(ei_hbm, et_hbm, tmpa, tmpb, segb, t0)

        for p in range(nper):
            c = cid * nper + p
            for j in range(NCH):
                pltpu.sync_copy(ei_hbm.at[pl.ds(t0 + j * CH, CH)], tmpa)
                gj = gidx[j]

                def gbody(i, _):
                    s = tmpa[pl.ds(i * 16, 16)]
                    gj[pl.ds(i * 16, 16)] = s * 8 + c
                    return 0
                lax.fori_loop(0, CHR, gbody, 0)
            for base, size in PASSES:
                _zero_rows(acc, zrow, sid * (ACC_ROWS // NT), ACC_ROWS // NT)
                plsc.subcore_barrier()
                for j in range(NCH):
                    sj = segb[j]

                    def mbody(i, _):
                        seg = sj[pl.ds(i * 16, 16)]
                        u = seg - base
                        valid = (u >= 0) & (u < size)
                        scur[pl.ds(i * 16, 16)] = jnp.where(
                            valid, u, size + (seg & 63))
                        return 0
                    lax.fori_loop(0, CHR, mbody, 0)
                    pltpu.sync_copy(x_hbm.at[gidx[j]], stg)
                    pltpu.sync_copy(stg, acc.at[scur], add=True)
                plsc.subcore_barrier()
                _repack_dump(acc, stg, obuf, s_out.at[c],
                             sid * (size // NT),
                             base // 8 + sid * (size // NT // 8),
                             size // NT)

    return pl.kernel(
        body,
        out_type=[jax.ShapeDtypeStruct((nc, N, 128), jnp.float32)],
        mesh=mesh,
        scratch_types=scratch,
        compiler_params=_sc_compiler_params(),
    )


def _tc_layer(nc, din, nb=2000):
    def body(s_ref, cnt_ref, x_ref, comp_ref, bases_ref, root_ref, bias_ref,
             o_ref):
        srep = 1.0 / jnp.maximum(cnt_ref[...], 1.0)
        acc = jnp.dot(x_ref[...], root_ref[...],
                      preferred_element_type=jnp.float32) + bias_ref[...]
        comp = comp_ref[...]
        for c in range(nc):
            bc = bases_ref[:, c * 16:(c + 1) * 16, :].reshape(NBASES, 16 * 128)
            wc = jnp.dot(comp, bc,
                         preferred_element_type=jnp.float32).reshape(R * 16,
                                                                     128)
            acc = acc + jnp.dot(s_ref[c] * srep, wc,
                                preferred_element_type=jnp.float32)
        o_ref[...] = jnp.where(acc > 0, acc, ALPHA_LRELU * acc)

    return pl.pallas_call(
        body,
        grid=(N // nb,),
        in_specs=[
            pl.BlockSpec((nc, nb, 128), lambda i: (0, i, 0)),
            pl.BlockSpec((nb, 128), lambda i: (i, 0)),
            pl.BlockSpec((nb, din), lambda i: (i, 0)),
            pl.BlockSpec((R, NBASES), lambda i: (0, 0)),
            pl.BlockSpec((NBASES, din, 128), lambda i: (0, 0, 0)),
            pl.BlockSpec((din, 128), lambda i: (0, 0)),
            pl.BlockSpec((1, 128), lambda i: (0, 0)),
        ],
        out_specs=pl.BlockSpec((nb, 128), lambda i: (i, 0)),
        out_shape=jax.ShapeDtypeStruct((N, 128), jnp.float32),
    )


def kernel(embeddings, edge_index, edge_type, comp1, bases1, root1, bias1,
           comp2, bases2, root2, bias2):
    ei_flat = edge_index.reshape(2 * E)
    cnt = _sc_counts()(ei_flat, edge_type)
    if isinstance(cnt, (list, tuple)):
        cnt = cnt[0]
    emb128 = jnp.concatenate(
        [embeddings, jnp.zeros((N, 64), jnp.float32)], axis=1)
    s1 = _sc_segsum(4)(emb128, ei_flat, edge_type)
    if isinstance(s1, (list, tuple)):
        s1 = s1[0]
    h = _tc_layer(4, 64)(s1, cnt, embeddings, comp1,
                         bases1, root1, bias1.reshape(1, 128))
    s2 = _sc_segsum(8)(h, ei_flat, edge_type)
    if isinstance(s2, (list, tuple)):
        s2 = s2[0]
    out = _tc_layer(8, 128)(s2, cnt, h, comp2, bases2,
                            root2, bias2.reshape(1, 128))
    return out
